# baseline (speedup 1.0000x reference)
import functools
import math

import jax
import jax.numpy as jnp
from jax import lax
from jax.experimental import pallas as pl
from jax.experimental.pallas import tpu as pltpu
from jax.experimental.pallas import tpu_sc as plsc

VOCAB = 1000
D = 128
N = 1024 * 200
LANES = 16

NUM_CORES = 2
NUM_SUBCORES = 16
NW = NUM_CORES * NUM_SUBCORES

CHUNK = 128
ROWS_PER_W = N // NW
CHUNKS_PER_W = ROWS_PER_W // CHUNK
RING = 3
DIST = 2
ROWS_T = 64

SCALE = math.sqrt(D)


def _sc_body(table_hbm, idx_hbm, pos_hbm, out_hbm,
             table_sh, accum_sh, idx_v, rows_v, iota_v,
             ssem, asem, gsem0, gsem1, gsem2, gsem3,
             psem0, psem1, psem2, psem3, osem0, osem1, osem2, osem3):
    sid = lax.axis_index("s")
    wid = sid * NUM_CORES + lax.axis_index("c")
    base = pl.multiple_of(wid * ROWS_PER_W, CHUNK)
    gsems = [gsem0, gsem1, gsem2, gsem3]
    psems = [psem0, psem1, psem2, psem3]
    osems = [osem0, osem1, osem2, osem3]

    trow = lax.min(sid * ROWS_T, VOCAB - ROWS_T)
    pltpu.sync_copy(table_hbm.at[pl.ds(trow, ROWS_T), :],
                    rows_v.at[0, pl.ds(0, ROWS_T)])

    def _scale(i, carry):
        for j in range(D // LANES):
            sl = pl.ds(j * LANES, LANES)
            rows_v[0, i, sl] = rows_v[0, i, sl] * SCALE
        return carry

    lax.fori_loop(0, ROWS_T, _scale, 0, unroll=2)

    pltpu.make_async_copy(rows_v.at[0, pl.ds(0, ROWS_T)],
                          table_sh.at[pl.ds(trow, ROWS_T), :], ssem).start()

    for k in range(CHUNK // LANES):
        iota_v[0, pl.ds(k * LANES, LANES)] = (lax.iota(jnp.int32, LANES)
                                              + k * LANES)
    pltpu.sync_copy(idx_hbm.at[pl.ds(base, ROWS_PER_W)], idx_v)

    pltpu.make_async_copy(rows_v.at[0, pl.ds(0, ROWS_T)],
                          table_sh.at[pl.ds(trow, ROWS_T), :], ssem).wait()
    plsc.subcore_barrier()

    def in_copies(c, b):
        start = pl.multiple_of(base + c * CHUNK, CHUNK)
        idx_sl = idx_v.at[pl.ds(pl.multiple_of(c * CHUNK, CHUNK), CHUNK)]
        g = pltpu.make_async_copy(table_sh.at[idx_sl], rows_v.at[b], gsems[b])
        p = pltpu.make_async_copy(pos_hbm.at[pl.ds(start, CHUNK), :],
                                  accum_sh.at[sid, b], psems[b])
        return g, p

    def out_copy(c, b):
        start = pl.multiple_of(base + c * CHUNK, CHUNK)
        return pltpu.make_async_copy(accum_sh.at[sid, b],
                                     out_hbm.at[pl.ds(start, CHUNK), :],
                                     osems[b])

    def body(c, b, owait, prefetch):
        g, p = in_copies(c, b)
        g.wait()
        p.wait()
        add = pltpu.async_copy(rows_v.at[b], accum_sh.at[sid, b].at[iota_v.at[0]],
                               asem, add=True)
        add.wait()
        pl.delay(3000)
        out_copy(c, b).start()
        if prefetch:
            b2 = (b + DIST) % RING
            if owait:
                out_copy(c + DIST - RING, b2).wait()
            g2, p2 = in_copies(c + DIST, b2)
            g2.start()
            p2.start()

    for b in range(DIST):
        g, p = in_copies(b, b)
        g.start()
        p.start()

    for c in range(DIST):
        body(c, c, owait=(c + DIST - RING >= 0), prefetch=True)

    def steady(i, carry):
        for k in range(RING):
            body(DIST + i * RING + k, (DIST + k) % RING,
                 owait=True, prefetch=True)
        return carry

    lax.fori_loop(0, (CHUNKS_PER_W - DIST - RING) // RING, steady, 0)

    for c in range(CHUNKS_PER_W - RING, CHUNKS_PER_W):
        body(c, c % RING, owait=True,
             prefetch=(c + DIST < CHUNKS_PER_W))

    for c in range(CHUNKS_PER_W - RING, CHUNKS_PER_W):
        out_copy(c, c % RING).wait()


@jax.jit
def _sc_embed(x_flat, position_flat, table):
    mesh = plsc.VectorSubcoreMesh(core_axis_name="c", subcore_axis_name="s")
    kern = functools.partial(
        pl.kernel,
        mesh=mesh,
        out_type=jax.ShapeDtypeStruct((N, D), jnp.float32),
        scratch_types=[
            pltpu.VMEM_SHARED((VOCAB, D), jnp.float32),
            pltpu.VMEM_SHARED((NUM_SUBCORES, RING, CHUNK, D), jnp.float32),
            pltpu.VMEM((ROWS_PER_W,), jnp.int32),
            pltpu.VMEM((RING, CHUNK, D), jnp.float32),
            pltpu.VMEM((1, CHUNK), jnp.int32),
            pltpu.SemaphoreType.DMA,
            pltpu.SemaphoreType.DMA,
            pltpu.SemaphoreType.DMA,
            pltpu.SemaphoreType.DMA,
            pltpu.SemaphoreType.DMA,
            pltpu.SemaphoreType.DMA,
            pltpu.SemaphoreType.DMA,
            pltpu.SemaphoreType.DMA,
            pltpu.SemaphoreType.DMA,
            pltpu.SemaphoreType.DMA,
            pltpu.SemaphoreType.DMA,
            pltpu.SemaphoreType.DMA,
            pltpu.SemaphoreType.DMA,
            pltpu.SemaphoreType.DMA,
        ],
    )(_sc_body)
    return kern(table, x_flat, position_flat)


def kernel(x, position, table):
    x_flat = x.reshape(N)
    pos_flat = position.reshape(N, D)
    out = _sc_embed(x_flat, pos_flat, table)
    return out.reshape(position.shape)


# speedup vs baseline: 2.9732x; 2.9732x over previous
import functools
import math

import jax
import jax.numpy as jnp
from jax import lax
from jax.experimental import pallas as pl
from jax.experimental.pallas import tpu as pltpu
from jax.experimental.pallas import tpu_sc as plsc

VOCAB = 1000
D = 128
N = 1024 * 200
LANES = 16

NUM_CORES = 2
NUM_SUBCORES = 16
NW = NUM_CORES * NUM_SUBCORES

CHUNK = 128
ROWS_PER_W = N // NW
CHUNKS_PER_W = ROWS_PER_W // CHUNK
NBUF = 2

SCALE = math.sqrt(D)


def _sc_body(table_hbm, idx_hbm, pos_hbm, out_hbm,
             table_sh, idx_v, rows_v, pos_v, out_v,
             tsem, gsem0, gsem1, psem0, psem1, osem0, osem1):
    sid = lax.axis_index("s")
    wid = sid * NUM_CORES + lax.axis_index("c")
    base = pl.multiple_of(wid * ROWS_PER_W, CHUNK)
    sems = [(gsem0, psem0, osem0), (gsem1, psem1, osem1)]

    @pl.when(sid == 0)
    def _():
        pltpu.make_async_copy(table_hbm, table_sh, tsem).start()

    pltpu.sync_copy(idx_hbm.at[pl.ds(base, ROWS_PER_W)], idx_v)

    @pl.when(sid == 0)
    def _():
        pltpu.make_async_copy(table_hbm, table_sh, tsem).wait()

    plsc.subcore_barrier()

    def in_copies(c, b):
        start = pl.multiple_of(base + c * CHUNK, CHUNK)
        idx_sl = idx_v.at[pl.ds(pl.multiple_of(c * CHUNK, CHUNK), CHUNK)]
        g = pltpu.make_async_copy(table_sh.at[idx_sl], rows_v.at[b],
                                  sems[b][0])
        p = pltpu.make_async_copy(pos_hbm.at[pl.ds(start, CHUNK), :],
                                  pos_v.at[b], sems[b][1])
        return g, p

    def out_copy(c, b):
        start = pl.multiple_of(base + c * CHUNK, CHUNK)
        return pltpu.make_async_copy(out_v.at[b],
                                     out_hbm.at[pl.ds(start, CHUNK), :],
                                     sems[b][2])

    def compute(b):
        @plsc.parallel_loop(0, CHUNK, 1, unroll=4)
        def row_body(i):
            for j in range(D // LANES):
                sl = pl.ds(j * LANES, LANES)
                out_v[b, i, sl] = rows_v[b, i, sl] * SCALE + pos_v[b, i, sl]

    for b in range(NBUF):
        g, p = in_copies(b, b)
        g.start()
        p.start()

    for b in range(NBUF):
        g, p = in_copies(b, b)
        g.wait()
        p.wait()
        compute(b)
        out_copy(b, b).start()
        g2, p2 = in_copies(b + NBUF, b)
        g2.start()
        p2.start()

    def steady(i, carry):
        for b in range(NBUF):
            c = i * NBUF + b
            g, p = in_copies(c, b)
            g.wait()
            p.wait()
            out_copy(c - NBUF, b).wait()
            compute(b)
            out_copy(c, b).start()
            g2, p2 = in_copies(c + NBUF, b)
            g2.start()
            p2.start()
        return carry

    lax.fori_loop(1, CHUNKS_PER_W // NBUF - 1, steady, 0)

    for b in range(NBUF):
        c = CHUNKS_PER_W - NBUF + b
        g, p = in_copies(c, b)
        g.wait()
        p.wait()
        out_copy(c - NBUF, b).wait()
        compute(b)
        out_copy(c, b).start()

    for b in range(NBUF):
        out_copy(CHUNKS_PER_W - NBUF + b, b).wait()


@jax.jit
def _sc_embed(x_flat, position_flat, table):
    mesh = plsc.VectorSubcoreMesh(core_axis_name="c", subcore_axis_name="s")
    kern = functools.partial(
        pl.kernel,
        mesh=mesh,
        out_type=jax.ShapeDtypeStruct((N, D), jnp.float32),
        scratch_types=[
            pltpu.VMEM_SHARED((VOCAB, D), jnp.float32),
            pltpu.VMEM((ROWS_PER_W,), jnp.int32),
            pltpu.VMEM((NBUF, CHUNK, D), jnp.float32),
            pltpu.VMEM((NBUF, CHUNK, D), jnp.float32),
            pltpu.VMEM((NBUF, CHUNK, D), jnp.float32),
            pltpu.SemaphoreType.DMA,
            pltpu.SemaphoreType.DMA,
            pltpu.SemaphoreType.DMA,
            pltpu.SemaphoreType.DMA,
            pltpu.SemaphoreType.DMA,
            pltpu.SemaphoreType.DMA,
            pltpu.SemaphoreType.DMA,
        ],
    )(_sc_body)
    return kern(table, x_flat, position_flat)


def kernel(x, position, table):
    x_flat = x.reshape(N)
    pos_flat = position.reshape(N, D)
    out = _sc_embed(x_flat, pos_flat, table)
    return out.reshape(position.shape)
